# baseline (speedup 1.0000x reference)
import jax
import jax.numpy as jnp
from jax.experimental import pallas as pl
from jax.experimental.pallas import tpu as pltpu

N_WIN2 = 49
TOPK = 4
DIM = 384


def _router_kernel(x0_ref, x1_ref, w_ref, b_ref, mask_ref, rs_ref):
    half = DIM // 2
    s = jax.lax.dot_general(
        x0_ref[0], w_ref[:, :half], (((0,), (1,)), ((), ())),
        preferred_element_type=jnp.float32)
    s = s + jax.lax.dot_general(
        x1_ref[0], w_ref[:, half:], (((0,), (1,)), ((), ())),
        preferred_element_type=jnp.float32)
    s = s + b_ref[0][None, :]

    m = jnp.max(s, axis=-1, keepdims=True)
    e = jnp.exp(s - m)
    rs = e / jnp.sum(e, axis=-1, keepdims=True)
    rs_ref[0] = rs

    work = s
    msk = jnp.zeros_like(s)
    for _ in range(TOPK):
        mx = jnp.max(work, axis=-1, keepdims=True)
        sel = work == mx
        msk = jnp.where(sel, 1.0, msk)
        work = jnp.where(sel, -jnp.inf, work)
    mask_ref[0] = msk


def kernel(x, W, b):
    B, C, H, Wd = x.shape
    HW = H * Wd
    x3 = x.reshape(B, C, HW)
    b2 = b.reshape(1, N_WIN2)
    out_shape = [
        jax.ShapeDtypeStruct((B, HW, N_WIN2), jnp.float32),
        jax.ShapeDtypeStruct((B, HW, N_WIN2), jnp.float32),
    ]
    half = C // 2
    mask, rs = pl.pallas_call(
        _router_kernel,
        grid=(B,),
        in_specs=[
            pl.BlockSpec((1, half, HW), lambda bb: (bb, 0, 0)),
            pl.BlockSpec((1, half, HW), lambda bb: (bb, 1, 0)),
            pl.BlockSpec((N_WIN2, C), lambda bb: (0, 0)),
            pl.BlockSpec((1, N_WIN2), lambda bb: (0, 0)),
        ],
        out_specs=[
            pl.BlockSpec((1, HW, N_WIN2), lambda bb: (bb, 0, 0)),
            pl.BlockSpec((1, HW, N_WIN2), lambda bb: (bb, 0, 0)),
        ],
        out_shape=out_shape,
        compiler_params=pltpu.CompilerParams(
            dimension_semantics=("parallel",)),
    )(x3, x3, W, b2)
    return (mask, rs)


# speedup vs baseline: 1.0721x; 1.0721x over previous
import jax
import jax.numpy as jnp
from jax.experimental import pallas as pl
from jax.experimental.pallas import tpu as pltpu

N_WIN2 = 49
TOPK = 4
DIM = 384


def _router_kernel(x_ref, w_ref, b_ref, mask_ref, rs_ref):
    s = jax.lax.dot_general(
        x_ref[0], w_ref[...], (((0,), (1,)), ((), ())),
        preferred_element_type=jnp.float32)
    s = s + b_ref[0][None, :]

    m = jnp.max(s, axis=-1, keepdims=True)
    e = jnp.exp(s - m)
    rs = e / jnp.sum(e, axis=-1, keepdims=True)
    rs_ref[0] = rs

    work = s
    msk = jnp.zeros_like(s)
    for _ in range(TOPK):
        mx = jnp.max(work, axis=-1, keepdims=True)
        sel = work == mx
        msk = jnp.where(sel, 1.0, msk)
        work = jnp.where(sel, -jnp.inf, work)
    mask_ref[0] = msk


def kernel(x, W, b):
    B, C, H, Wd = x.shape
    HW = H * Wd
    x3 = x.reshape(B, C, HW)
    b2 = b.reshape(1, N_WIN2)
    out_shape = [
        jax.ShapeDtypeStruct((B, HW, N_WIN2), jnp.float32),
        jax.ShapeDtypeStruct((B, HW, N_WIN2), jnp.float32),
    ]
    mask, rs = pl.pallas_call(
        _router_kernel,
        grid=(B,),
        in_specs=[
            pl.BlockSpec((1, C, HW), lambda bb: (bb, 0, 0)),
            pl.BlockSpec((N_WIN2, C), lambda bb: (0, 0)),
            pl.BlockSpec((1, N_WIN2), lambda bb: (0, 0)),
        ],
        out_specs=[
            pl.BlockSpec((1, HW, N_WIN2), lambda bb: (bb, 0, 0)),
            pl.BlockSpec((1, HW, N_WIN2), lambda bb: (bb, 0, 0)),
        ],
        out_shape=out_shape,
        compiler_params=pltpu.CompilerParams(
            dimension_semantics=("parallel",)),
    )(x3, W, b2)
    return (mask, rs)
